# baseline (device time: 151246 ns/iter reference)
import jax
import jax.numpy as jnp
from jax import lax
from jax.experimental import pallas as pl
from jax.experimental.pallas import tpu as pltpu

N_DEV = 4
_GELU_C = 0.7978845608028654


def _gelu(y):
    return 0.5 * y * (1.0 + jnp.tanh(_GELU_C * (y + 0.044715 * y * y * y)))


def kernel(x, w_mat):
    m_per, k = x.shape
    _, n_per = w_mat.shape

    def body(x_ref, w_ref, out_ref, comm_ref, send_sems, recv_sems):
        my = lax.axis_index("i")
        left = lax.rem(my - 1 + N_DEV, N_DEV)
        right = lax.rem(my + 1, N_DEV)

        barrier_sem = pltpu.get_barrier_semaphore()
        for nbr in (left, right):
            pl.semaphore_signal(
                barrier_sem, inc=1,
                device_id=(nbr,), device_id_type=pl.DeviceIdType.MESH,
            )
        pl.semaphore_wait(barrier_sem, 2)

        comm_ref[0, :, :] = x_ref[:, :]

        rdmas = []
        for h in range(N_DEV - 1):
            rdma = pltpu.make_async_remote_copy(
                src_ref=comm_ref.at[h],
                dst_ref=comm_ref.at[h + 1],
                send_sem=send_sems.at[h],
                recv_sem=recv_sems.at[h],
                device_id=(right,),
                device_id_type=pl.DeviceIdType.MESH,
            )
            rdma.start()
            rdmas.append(rdma)
            origin = lax.rem(my - h + N_DEV, N_DEV)
            y = jnp.dot(comm_ref[h, :, :], w_ref[:, :],
                        preferred_element_type=jnp.float32)
            out_ref[pl.ds(origin * m_per, m_per), :] = _gelu(y)
            rdma.wait_recv()

        origin = lax.rem(my - (N_DEV - 1) + N_DEV, N_DEV)
        y = jnp.dot(comm_ref[N_DEV - 1, :, :], w_ref[:, :],
                    preferred_element_type=jnp.float32)
        out_ref[pl.ds(origin * m_per, m_per), :] = _gelu(y)

        for rdma in rdmas:
            rdma.wait_send()

    return pl.pallas_call(
        body,
        out_shape=jax.ShapeDtypeStruct((N_DEV * m_per, n_per), jnp.float32),
        in_specs=[
            pl.BlockSpec(memory_space=pltpu.VMEM),
            pl.BlockSpec(memory_space=pltpu.VMEM),
        ],
        out_specs=pl.BlockSpec(memory_space=pltpu.VMEM),
        scratch_shapes=[
            pltpu.VMEM((N_DEV, m_per, k), jnp.float32),
            pltpu.SemaphoreType.DMA((N_DEV - 1,)),
            pltpu.SemaphoreType.DMA((N_DEV - 1,)),
        ],
        compiler_params=pltpu.CompilerParams(collective_id=0),
    )(x, w_mat)


# device time: 84044 ns/iter; 1.7996x vs baseline; 1.7996x over previous
import jax
import jax.numpy as jnp
from jax import lax
from jax.experimental import pallas as pl
from jax.experimental.pallas import tpu as pltpu

N_DEV = 4
_GELU_C = 0.7978845608028654


def _gelu(y):
    return 0.5 * y * (1.0 + jnp.tanh(_GELU_C * (y + 0.044715 * y * y * y)))


def kernel(x, w_mat):
    m_per, k = x.shape
    _, n_per = w_mat.shape
    m_half = m_per // 2

    def body(x_ref, w_ref, out_ref,
             comm_a, comm_b, send_a, recv_a, send_b, recv_b):
        my = lax.axis_index("i")
        left = lax.rem(my - 1 + N_DEV, N_DEV)
        right = lax.rem(my + 1, N_DEV)

        barrier_sem = pltpu.get_barrier_semaphore()
        for nbr in (left, right):
            pl.semaphore_signal(
                barrier_sem, inc=1,
                device_id=(nbr,), device_id_type=pl.DeviceIdType.MESH,
            )
        pl.semaphore_wait(barrier_sem, 2)

        comm_a[0, :, :] = x_ref[:m_half, :]
        comm_b[0, :, :] = x_ref[m_half:, :]

        def compute(slot, h):
            origin_a = lax.rem(my - h + N_DEV, N_DEV)
            origin_b = lax.rem(my + h, N_DEV)
            ya = jnp.dot(comm_a[slot, :, :], w_ref[:, :],
                         preferred_element_type=jnp.float32)
            out_ref[pl.ds(origin_a * m_per, m_half), :] = _gelu(ya)
            yb = jnp.dot(comm_b[slot, :, :], w_ref[:, :],
                         preferred_element_type=jnp.float32)
            out_ref[pl.ds(origin_b * m_per + m_half, m_half), :] = _gelu(yb)

        rdmas = []
        for h in range(N_DEV - 1):
            rdma_a = pltpu.make_async_remote_copy(
                src_ref=comm_a.at[h],
                dst_ref=comm_a.at[h + 1],
                send_sem=send_a.at[h],
                recv_sem=recv_a.at[h],
                device_id=(right,),
                device_id_type=pl.DeviceIdType.MESH,
            )
            rdma_b = pltpu.make_async_remote_copy(
                src_ref=comm_b.at[h],
                dst_ref=comm_b.at[h + 1],
                send_sem=send_b.at[h],
                recv_sem=recv_b.at[h],
                device_id=(left,),
                device_id_type=pl.DeviceIdType.MESH,
            )
            rdma_a.start()
            rdma_b.start()
            rdmas += [rdma_a, rdma_b]
            compute(h, h)
            rdma_a.wait_recv()
            rdma_b.wait_recv()

        compute(N_DEV - 1, N_DEV - 1)

        for rdma in rdmas:
            rdma.wait_send()

    return pl.pallas_call(
        body,
        out_shape=jax.ShapeDtypeStruct((N_DEV * m_per, n_per), jnp.float32),
        in_specs=[
            pl.BlockSpec(memory_space=pltpu.VMEM),
            pl.BlockSpec(memory_space=pltpu.VMEM),
        ],
        out_specs=pl.BlockSpec(memory_space=pltpu.VMEM),
        scratch_shapes=[
            pltpu.VMEM((N_DEV, m_half, k), jnp.float32),
            pltpu.VMEM((N_DEV, m_half, k), jnp.float32),
            pltpu.SemaphoreType.DMA((N_DEV - 1,)),
            pltpu.SemaphoreType.DMA((N_DEV - 1,)),
            pltpu.SemaphoreType.DMA((N_DEV - 1,)),
            pltpu.SemaphoreType.DMA((N_DEV - 1,)),
        ],
        compiler_params=pltpu.CompilerParams(collective_id=0),
    )(x, w_mat)


# device time: 80439 ns/iter; 1.8803x vs baseline; 1.0448x over previous
import jax
import jax.numpy as jnp
from jax import lax
from jax.experimental import pallas as pl
from jax.experimental.pallas import tpu as pltpu

N_DEV = 4
N_HOP = N_DEV - 1
N_SUB = 2
_GELU_C = 0.7978845608028654


def _gelu(y):
    return 0.5 * y * (1.0 + jnp.tanh(_GELU_C * (y + 0.044715 * y * y * y)))


def kernel(x, w_mat):
    m_per, k = x.shape
    _, n_per = w_mat.shape
    m_half = m_per // 2
    m_sub = m_half // N_SUB

    def body(x_ref, w_ref, out_ref,
             comm_a, comm_b, send_a, recv_a, send_b, recv_b):
        my = lax.axis_index("i")
        left = lax.rem(my - 1 + N_DEV, N_DEV)
        right = lax.rem(my + 1, N_DEV)

        barrier_sem = pltpu.get_barrier_semaphore()
        for nbr in (left, right):
            pl.semaphore_signal(
                barrier_sem, inc=1,
                device_id=(nbr,), device_id_type=pl.DeviceIdType.MESH,
            )
        pl.semaphore_wait(barrier_sem, 2)

        def make_rdma(ring_comm, sems_pair, dev, h, j):
            if h == 0:
                base = 0 if ring_comm is comm_a else m_half
                src = x_ref.at[pl.ds(base + j * m_sub, m_sub)]
            else:
                src = ring_comm.at[h - 1, pl.ds(j * m_sub, m_sub)]
            send, recv = sems_pair
            return pltpu.make_async_remote_copy(
                src_ref=src,
                dst_ref=ring_comm.at[h, pl.ds(j * m_sub, m_sub)],
                send_sem=send.at[h, j],
                recv_sem=recv.at[h, j],
                device_id=(dev,),
                device_id_type=pl.DeviceIdType.MESH,
            )

        def compute(slot):
            origin_a = lax.rem(my - slot - 1 + N_DEV, N_DEV)
            origin_b = lax.rem(my + slot + 1, N_DEV)
            ya = jnp.dot(comm_a[slot, :, :], w_ref[:, :],
                         preferred_element_type=jnp.float32)
            out_ref[pl.ds(origin_a * m_per, m_half), :] = _gelu(ya)
            yb = jnp.dot(comm_b[slot, :, :], w_ref[:, :],
                         preferred_element_type=jnp.float32)
            out_ref[pl.ds(origin_b * m_per + m_half, m_half), :] = _gelu(yb)

        rdmas = {}
        for ring, sems, dev in (
            (comm_a, (send_a, recv_a), right),
            (comm_b, (send_b, recv_b), left),
        ):
            for j in range(N_SUB):
                r = make_rdma(ring, sems, dev, 0, j)
                r.start()
                rdmas[(id(ring), 0, j)] = r

        y = jnp.dot(x_ref[:, :], w_ref[:, :],
                    preferred_element_type=jnp.float32)
        out_ref[pl.ds(my * m_per, m_per), :] = _gelu(y)

        for h in range(1, N_HOP):
            for j in range(N_SUB):
                for ring, sems, dev in (
                    (comm_a, (send_a, recv_a), right),
                    (comm_b, (send_b, recv_b), left),
                ):
                    rdmas[(id(ring), h - 1, j)].wait_recv()
                    r = make_rdma(ring, sems, dev, h, j)
                    r.start()
                    rdmas[(id(ring), h, j)] = r
            compute(h - 1)

        for j in range(N_SUB):
            rdmas[(id(comm_a), N_HOP - 1, j)].wait_recv()
            rdmas[(id(comm_b), N_HOP - 1, j)].wait_recv()
        compute(N_HOP - 1)

        for r in rdmas.values():
            r.wait_send()

    return pl.pallas_call(
        body,
        out_shape=jax.ShapeDtypeStruct((N_DEV * m_per, n_per), jnp.float32),
        in_specs=[
            pl.BlockSpec(memory_space=pltpu.VMEM),
            pl.BlockSpec(memory_space=pltpu.VMEM),
        ],
        out_specs=pl.BlockSpec(memory_space=pltpu.VMEM),
        scratch_shapes=[
            pltpu.VMEM((N_HOP, m_half, k), jnp.float32),
            pltpu.VMEM((N_HOP, m_half, k), jnp.float32),
            pltpu.SemaphoreType.DMA((N_HOP, N_SUB)),
            pltpu.SemaphoreType.DMA((N_HOP, N_SUB)),
            pltpu.SemaphoreType.DMA((N_HOP, N_SUB)),
            pltpu.SemaphoreType.DMA((N_HOP, N_SUB)),
        ],
        compiler_params=pltpu.CompilerParams(collective_id=0),
    )(x, w_mat)
